# baseline (speedup 1.0000x reference)
import functools
import math

import jax
import jax.numpy as jnp
from jax import lax
from jax.experimental import pallas as pl
from jax.experimental.pallas import tpu as pltpu
from jax.experimental.pallas import tpu_sc as plsc

_VOCAB = 1000000
_D = 64
_SCALE = math.sqrt(_D)

_NB = 4096 * 200
_NC = 2
_NS = 16
_NW = _NC * _NS
_PER_W = _NB // _NW
_CH = 128
_NCH = _PER_W // _CH

_mesh = plsc.VectorSubcoreMesh(core_axis_name="c", subcore_axis_name="s")


@functools.partial(
    pl.kernel,
    mesh=_mesh,
    out_type=jax.ShapeDtypeStruct((_NB, _D), jnp.float32),
    scratch_types=[
        pltpu.VMEM((_NCH, _CH), jnp.int32),
        pltpu.VMEM((_CH, _D), jnp.float32),
        pltpu.SemaphoreType.DMA,
    ],
    compiler_params=pltpu.CompilerParams(use_tc_tiling_on_sc=False),
)
def _embed_sc(idx_hbm, table_hbm, out_hbm, idx_v, rows_v, sem):
    wid = lax.axis_index("s") * _NC + lax.axis_index("c")
    base = wid * _PER_W
    pltpu.sync_copy(idx_hbm.at[wid], idx_v)

    def chunk_body(j, carry):
        pltpu.async_copy(table_hbm.at[idx_v.at[j]], rows_v, sem).wait()

        def row_body(i, c2):
            for c in range(_D // 16):
                sl = pl.ds(c * 16, 16)
                rows_v[i, sl] = rows_v[i, sl] * _SCALE
            return c2

        lax.fori_loop(0, _CH, row_body, 0, unroll=2)

        pltpu.sync_copy(rows_v, out_hbm.at[pl.ds(base + j * _CH, _CH)])
        return carry

    lax.fori_loop(0, _NCH, chunk_body, 0)


def kernel(input_word, embed_weight):
    idx = jnp.reshape(input_word.astype(jnp.int32), (_NW, _NCH, _CH))
    out = _embed_sc(idx, embed_weight)
    return jnp.reshape(out, (*input_word.shape, _D))


# speedup vs baseline: 1.0546x; 1.0546x over previous
import functools
import math

import jax
import jax.numpy as jnp
from jax import lax
from jax.experimental import pallas as pl
from jax.experimental.pallas import tpu as pltpu
from jax.experimental.pallas import tpu_sc as plsc

_VOCAB = 1000000
_D = 64
_SCALE = math.sqrt(_D)

_NB = 4096 * 200
_NC = 2
_NS = 16
_NW = _NC * _NS
_PER_W = _NB // _NW
_CH = 128
_NCH = _PER_W // _CH
_K = 4
_NSUP = _NCH // _K

_mesh = plsc.VectorSubcoreMesh(core_axis_name="c", subcore_axis_name="s")


@functools.partial(
    pl.kernel,
    mesh=_mesh,
    out_type=jax.ShapeDtypeStruct((_NB, _D), jnp.float32),
    scratch_types=[
        pltpu.VMEM((_NCH, _CH), jnp.int32),
        pltpu.VMEM((_K, _CH, _D), jnp.float32),
        pltpu.VMEM((_K, _CH, _D), jnp.float32),
        pltpu.SemaphoreType.DMA((_K,)),
        pltpu.SemaphoreType.DMA((_K,)),
    ],
    compiler_params=pltpu.CompilerParams(use_tc_tiling_on_sc=False),
)
def _embed_sc(idx_hbm, table_hbm, out_hbm, idx_v, gbuf, sbuf, gsem, ssem):
    wid = lax.axis_index("s") * _NC + lax.axis_index("c")
    base = wid * _PER_W
    pltpu.sync_copy(idx_hbm.at[wid], idx_v)

    def gather_start(j, b):
        pltpu.make_async_copy(
            table_hbm.at[idx_v.at[j]], gbuf.at[b], gsem.at[b]
        ).start()

    def gather_wait(j, b):
        pltpu.make_async_copy(
            table_hbm.at[idx_v.at[j]], gbuf.at[b], gsem.at[b]
        ).wait()

    def store_start(j, b):
        pltpu.make_async_copy(
            sbuf.at[b], out_hbm.at[pl.ds(base + j * _CH, _CH)], ssem.at[b]
        ).start()

    def store_wait(j, b):
        pltpu.make_async_copy(
            sbuf.at[b], out_hbm.at[pl.ds(base + j * _CH, _CH)], ssem.at[b]
        ).wait()

    for b in range(_K):
        gather_start(b, b)

    def superstep(s, carry):
        for b in range(_K):
            j = s * _K + b
            gather_wait(j, b)

            @pl.when(s > 0)
            def _():
                store_wait(j - _K, b)

            def row_body(i, c2):
                for c in range(_D // 16):
                    sl = pl.ds(c * 16, 16)
                    sbuf[b, i, sl] = gbuf[b, i, sl] * _SCALE
                return c2

            lax.fori_loop(0, _CH, row_body, 0, unroll=4)

            @pl.when(s < _NSUP - 1)
            def _():
                gather_start(j + _K, b)

            store_start(j, b)
        return carry

    lax.fori_loop(0, _NSUP, superstep, 0)

    for b in range(_K):
        store_wait((_NSUP - 1) * _K + b, b)


def kernel(input_word, embed_weight):
    idx = jnp.reshape(input_word.astype(jnp.int32), (_NW, _NCH, _CH))
    out = _embed_sc(idx, embed_weight)
    return jnp.reshape(out, (*input_word.shape, _D))
